# Initial kernel scaffold; baseline (speedup 1.0000x reference)
import jax
import jax.numpy as jnp
from jax.experimental import pallas as pl


def kernel(edge_costs, edge_counter, t12_costs, t13_costs, t23_costs, tri_corr_12, tri_corr_13, tri_corr_23, W1, b1, W2, b2):
    raise NotImplementedError("write your pallas kernel here")


# speedup vs baseline: 47.2117x; 47.2117x over previous
import functools

import jax
import jax.numpy as jnp
from jax import lax
from jax.experimental import pallas as pl
from jax.experimental.pallas import tpu as pltpu
from jax.experimental.pallas import tpu_sc as plsc

E = 1_600_000
NUM_WORKERS = 32
REF_PER = E // NUM_WORKERS
CHUNK = 10_000
NCH = REF_PER // CHUNK
SLICE = E // 16
ZB = 10_000


def _sc_scatter(c12, c13, c23, t12, t13, t23, s_out, r_out,
                acc_sh, idx_v, val_v, fill_v):
    core = lax.axis_index("c")
    sid = lax.axis_index("s")
    wid = sid * 2 + core

    def fill(value):
        @pl.loop(0, ZB // 16)
        def _(i):
            off = pl.multiple_of(i * 16, 16)
            fill_v[pl.ds(off, 16)] = jnp.full((16,), value, jnp.float32)

    def clear_acc():
        @pl.loop(0, SLICE // ZB)
        def _(j):
            off = pl.multiple_of(sid * SLICE + j * ZB, 8)
            pltpu.sync_copy(fill_v, acc_sh.at[pl.ds(off, ZB)])

    def scatter_pass(vals_from_hbm):
        for idx_hbm, val_hbm in ((c12, t12), (c13, t13), (c23, t23)):
            @pl.loop(0, NCH)
            def _(j):
                off = pl.multiple_of(wid * REF_PER + j * CHUNK, 8)
                pltpu.sync_copy(idx_hbm.at[pl.ds(off, CHUNK)], idx_v)
                if vals_from_hbm:
                    pltpu.sync_copy(val_hbm.at[pl.ds(off, CHUNK)], val_v)
                    pltpu.sync_copy(val_v, acc_sh.at[idx_v], add=True)
                else:
                    pltpu.sync_copy(fill_v, acc_sh.at[idx_v], add=True)

    def dump(out_hbm):
        @pl.loop(0, SLICE // CHUNK)
        def _(j):
            off = pl.multiple_of(sid * SLICE + j * CHUNK, 8)
            oout = pl.multiple_of(core * E + sid * SLICE + j * CHUNK, 8)
            pltpu.sync_copy(acc_sh.at[pl.ds(off, CHUNK)], val_v)
            pltpu.sync_copy(val_v, out_hbm.at[pl.ds(oout, CHUNK)])

    fill(0.0)
    clear_acc()
    plsc.subcore_barrier()
    scatter_pass(vals_from_hbm=True)
    plsc.subcore_barrier()
    dump(s_out)
    plsc.subcore_barrier()
    fill(0.0)
    clear_acc()
    plsc.subcore_barrier()
    fill(1.0)
    scatter_pass(vals_from_hbm=False)
    plsc.subcore_barrier()
    dump(r_out)


_sc_call = functools.partial(
    pl.kernel,
    out_type=(
        jax.ShapeDtypeStruct((2 * E,), jnp.float32),
        jax.ShapeDtypeStruct((2 * E,), jnp.float32),
    ),
    mesh=plsc.VectorSubcoreMesh(core_axis_name="c", subcore_axis_name="s"),
    scratch_types=[
        pltpu.VMEM_SHARED((E,), jnp.float32),
        pltpu.VMEM((CHUNK,), jnp.int32),
        pltpu.VMEM((CHUNK,), jnp.float32),
        pltpu.VMEM((ZB,), jnp.float32),
    ],
)(_sc_scatter)


ROWS = 1_000
COLS = 1_600
BLK = 200


def _combine_body(ec_ref, cnt_ref, s_ref, r_ref, out_ref):
    ec = ec_ref[...]
    cnt_i = cnt_ref[...]
    cnt = jnp.maximum(cnt_i.astype(jnp.float32), 1.0)
    inv = 1.0 / cnt
    s = s_ref[0] + s_ref[1]
    r = r_ref[0] + r_ref[1]
    base = jnp.where(cnt_i > 0, 0.0, ec)
    out_ref[...] = base + s * inv + r * ec * inv * inv


def _combine(ec, cnt, s, r):
    ec2 = ec.reshape(ROWS, COLS)
    cnt2 = cnt.reshape(ROWS, COLS)
    s3 = s.reshape(2, ROWS, COLS)
    r3 = r.reshape(2, ROWS, COLS)
    grid = ROWS // BLK
    out = pl.pallas_call(
        _combine_body,
        out_shape=jax.ShapeDtypeStruct((ROWS, COLS), jnp.float32),
        grid=(grid,),
        in_specs=[
            pl.BlockSpec((BLK, COLS), lambda i: (i, 0)),
            pl.BlockSpec((BLK, COLS), lambda i: (i, 0)),
            pl.BlockSpec((2, BLK, COLS), lambda i: (0, i, 0)),
            pl.BlockSpec((2, BLK, COLS), lambda i: (0, i, 0)),
        ],
        out_specs=pl.BlockSpec((BLK, COLS), lambda i: (i, 0)),
    )(ec2, cnt2, s3, r3)
    return out.reshape(E)


def kernel(edge_costs, edge_counter, t12_costs, t13_costs, t23_costs,
           tri_corr_12, tri_corr_13, tri_corr_23, W1, b1, W2, b2):
    s, r = _sc_call(tri_corr_12, tri_corr_13, tri_corr_23,
                    t12_costs, t13_costs, t23_costs)
    ec = _combine(edge_costs, edge_counter, s, r)
    z = jnp.zeros_like(t12_costs)
    return ec, z, z, z


# speedup vs baseline: 68.3119x; 1.4469x over previous
import functools

import jax
import jax.numpy as jnp
from jax import lax
from jax.experimental import pallas as pl
from jax.experimental.pallas import tpu as pltpu
from jax.experimental.pallas import tpu_sc as plsc

E = 1_600_000
NUM_TILES = 16
TILE_REF = E // NUM_TILES
CHUNK = 4_000
SLICE = E // NUM_TILES


def _sc_scatter(c12, c13, c23, t12, t13, t23, s_out, r_out,
                acc_sh, idx_a, idx_b, val_a, val_b, ones_v, zero_v,
                sem_la, sem_lb, sem_va, sem_vb, sem_sa, sem_sb):
    core = lax.axis_index("c")
    sid = lax.axis_index("s")

    idx_bufs = (idx_a, idx_b)
    val_bufs = (val_a, val_b)
    lsem = (sem_la, sem_lb)
    vsem = (sem_va, sem_vb)
    ssem = (sem_sa, sem_sb)

    @pl.loop(0, CHUNK // 16)
    def _(i):
        off = pl.multiple_of(i * 16, 16)
        zero_v[pl.ds(off, 16)] = jnp.zeros((16,), jnp.float32)
        ones_v[pl.ds(off, 16)] = jnp.full((16,), 1.0, jnp.float32)

    zdescs = []
    for k in range(SLICE // CHUNK):
        off = pl.multiple_of(sid * SLICE + k * CHUNK, 8)
        zdescs.append(pltpu.async_copy(
            zero_v, acc_sh.at[pl.ds(off, CHUNK)], ssem[k % 2]))
    for d in zdescs:
        d.wait()
    plsc.subcore_barrier()

    chunks = []
    for idx_hbm, val_hbm in ((c12, t12), (c13, t13), (c23, t23)):
        for j in range(TILE_REF // CHUNK):
            chunks.append((idx_hbm, val_hbm,
                           sid * TILE_REF + j * CHUNK))
    n = len(chunks)

    def scatter_loop(with_vals):
        def load(i):
            b = i % 2
            ih, vh, off = chunks[i]
            offc = pl.multiple_of(off, 8)
            di = pltpu.async_copy(ih.at[pl.ds(offc, CHUNK)], idx_bufs[b],
                                  lsem[b])
            dv = None
            if with_vals:
                dv = pltpu.async_copy(vh.at[pl.ds(offc, CHUNK)], val_bufs[b],
                                      vsem[b])
            return di, dv

        loads = [None, None]
        scat = None
        loads[0] = load(0)
        for i in range(n):
            b = i % 2
            di, dv = loads[b]
            di.wait()
            if dv is not None:
                dv.wait()
            if scat is not None:
                scat.wait()
            if i + 1 < n:
                loads[1 - b] = load(i + 1)
            src = val_bufs[b] if with_vals else ones_v
            scat = pltpu.async_copy(src, acc_sh.at[idx_bufs[b]],
                                    ssem[b], add=True)
        scat.wait()

    @pl.when(core == 0)
    def _():
        scatter_loop(with_vals=True)

    @pl.when(core == 1)
    def _():
        scatter_loop(with_vals=False)

    plsc.subcore_barrier()

    def dump(out_hbm):
        descs = [None, None]
        for k in range(SLICE // CHUNK):
            b = k % 2
            if descs[b] is not None:
                descs[b].wait()
            off = pl.multiple_of(sid * SLICE + k * CHUNK, 8)
            pltpu.sync_copy(acc_sh.at[pl.ds(off, CHUNK)], val_bufs[b])
            descs[b] = pltpu.async_copy(val_bufs[b],
                                        out_hbm.at[pl.ds(off, CHUNK)],
                                        ssem[b])
        for d in descs:
            if d is not None:
                d.wait()

    @pl.when(core == 0)
    def _():
        dump(s_out)

    @pl.when(core == 1)
    def _():
        dump(r_out)


_sc_call = functools.partial(
    pl.kernel,
    out_type=(
        jax.ShapeDtypeStruct((E,), jnp.float32),
        jax.ShapeDtypeStruct((E,), jnp.float32),
    ),
    mesh=plsc.VectorSubcoreMesh(core_axis_name="c", subcore_axis_name="s"),
    scratch_types=[
        pltpu.VMEM_SHARED((E,), jnp.float32),
        pltpu.VMEM((CHUNK,), jnp.int32),
        pltpu.VMEM((CHUNK,), jnp.int32),
        pltpu.VMEM((CHUNK,), jnp.float32),
        pltpu.VMEM((CHUNK,), jnp.float32),
        pltpu.VMEM((CHUNK,), jnp.float32),
        pltpu.VMEM((CHUNK,), jnp.float32),
        pltpu.SemaphoreType.DMA,
        pltpu.SemaphoreType.DMA,
        pltpu.SemaphoreType.DMA,
        pltpu.SemaphoreType.DMA,
        pltpu.SemaphoreType.DMA,
        pltpu.SemaphoreType.DMA,
    ],
)(_sc_scatter)


ROWS = 1_000
COLS = 1_600
BLK = 200


def _combine_body(ec_ref, cnt_ref, s_ref, r_ref, out_ref):
    ec = ec_ref[...]
    cnt_i = cnt_ref[...]
    cnt = jnp.maximum(cnt_i.astype(jnp.float32), 1.0)
    inv = 1.0 / cnt
    base = jnp.where(cnt_i > 0, 0.0, ec)
    out_ref[...] = base + s_ref[...] * inv + r_ref[...] * ec * inv * inv


def _combine(ec, cnt, s, r):
    grid = ROWS // BLK
    out = pl.pallas_call(
        _combine_body,
        out_shape=jax.ShapeDtypeStruct((ROWS, COLS), jnp.float32),
        grid=(grid,),
        in_specs=[
            pl.BlockSpec((BLK, COLS), lambda i: (i, 0)),
            pl.BlockSpec((BLK, COLS), lambda i: (i, 0)),
            pl.BlockSpec((BLK, COLS), lambda i: (i, 0)),
            pl.BlockSpec((BLK, COLS), lambda i: (i, 0)),
        ],
        out_specs=pl.BlockSpec((BLK, COLS), lambda i: (i, 0)),
    )(ec.reshape(ROWS, COLS), cnt.reshape(ROWS, COLS),
      s.reshape(ROWS, COLS), r.reshape(ROWS, COLS))
    return out.reshape(E)


def kernel(edge_costs, edge_counter, t12_costs, t13_costs, t23_costs,
           tri_corr_12, tri_corr_13, tri_corr_23, W1, b1, W2, b2):
    s, r = _sc_call(tri_corr_12, tri_corr_13, tri_corr_23,
                    t12_costs, t13_costs, t23_costs)
    ec = _combine(edge_costs, edge_counter, s, r)
    z = jnp.zeros_like(t12_costs)
    return ec, z, z, z


# speedup vs baseline: 68.9681x; 1.0096x over previous
import functools

import jax
import jax.numpy as jnp
from jax import lax
from jax.experimental import pallas as pl
from jax.experimental.pallas import tpu as pltpu
from jax.experimental.pallas import tpu_sc as plsc

E = 1_600_000
NUM_TILES = 16
TILE_REF = E // NUM_TILES
CHUNK = 4_000
SLICE = E // NUM_TILES


def _sc_scatter(c12, c13, c23, t12, t13, t23, s_out, r_out,
                acc_sh, idx_a, idx_b, val_a, val_b, ones_v, zero_v,
                sem_la, sem_lb, sem_va, sem_vb, sem_sa, sem_sb):
    core = lax.axis_index("c")
    sid = lax.axis_index("s")

    idx_bufs = (idx_a, idx_b)
    val_bufs = (val_a, val_b)
    lsem = (sem_la, sem_lb)
    vsem = (sem_va, sem_vb)
    ssem = (sem_sa, sem_sb)

    @pl.loop(0, CHUNK // 16)
    def _(i):
        off = pl.multiple_of(i * 16, 16)
        zero_v[pl.ds(off, 16)] = jnp.zeros((16,), jnp.float32)
        ones_v[pl.ds(off, 16)] = jnp.full((16,), 1.0, jnp.float32)

    chunks = []
    for idx_hbm, val_hbm in ((c12, t12), (c13, t13), (c23, t23)):
        for j in range(TILE_REF // CHUNK):
            chunks.append((idx_hbm, val_hbm,
                           sid * TILE_REF + j * CHUNK))
    n = len(chunks)

    def load(i, with_vals):
        b = i % 2
        ih, vh, off = chunks[i]
        offc = pl.multiple_of(off, 8)
        di = pltpu.async_copy(ih.at[pl.ds(offc, CHUNK)], idx_bufs[b],
                              lsem[b])
        dv = None
        if with_vals:
            dv = pltpu.async_copy(vh.at[pl.ds(offc, CHUNK)], val_bufs[b],
                                  vsem[b])
        return di, dv

    preloads = [load(0, True), load(1, True)]
    zdescs = []
    for k in range(SLICE // CHUNK):
        off = pl.multiple_of(sid * SLICE + k * CHUNK, 8)
        zdescs.append(pltpu.async_copy(
            zero_v, acc_sh.at[pl.ds(off, CHUNK)], ssem[k % 2]))
    for d in zdescs:
        d.wait()
    plsc.subcore_barrier()

    def scatter_loop(with_vals):
        loads = list(preloads)
        scats = [None, None]
        for i in range(n):
            b = i % 2
            di, dv = loads[b]
            di.wait()
            if dv is not None:
                dv.wait()
            if 2 <= i + 1 < n:
                if scats[1 - b] is not None:
                    scats[1 - b].wait()
                    scats[1 - b] = None
                loads[1 - b] = load(i + 1, with_vals)
            src = val_bufs[b] if with_vals else ones_v
            scats[b] = pltpu.async_copy(src, acc_sh.at[idx_bufs[b]],
                                        ssem[b], add=True)
        for d in scats:
            if d is not None:
                d.wait()

    @pl.when(core == 0)
    def _():
        scatter_loop(with_vals=True)

    @pl.when(core == 1)
    def _():
        scatter_loop(with_vals=False)

    plsc.subcore_barrier()

    def dump(out_hbm):
        descs = [None, None]
        for k in range(SLICE // CHUNK):
            b = k % 2
            if descs[b] is not None:
                descs[b].wait()
            off = pl.multiple_of(sid * SLICE + k * CHUNK, 8)
            pltpu.sync_copy(acc_sh.at[pl.ds(off, CHUNK)], val_bufs[b])
            descs[b] = pltpu.async_copy(val_bufs[b],
                                        out_hbm.at[pl.ds(off, CHUNK)],
                                        ssem[b])
        for d in descs:
            if d is not None:
                d.wait()

    @pl.when(core == 0)
    def _():
        dump(s_out)

    @pl.when(core == 1)
    def _():
        dump(r_out)


_sc_call = functools.partial(
    pl.kernel,
    out_type=(
        jax.ShapeDtypeStruct((E,), jnp.float32),
        jax.ShapeDtypeStruct((E,), jnp.float32),
    ),
    mesh=plsc.VectorSubcoreMesh(core_axis_name="c", subcore_axis_name="s"),
    scratch_types=[
        pltpu.VMEM_SHARED((E,), jnp.float32),
        pltpu.VMEM((CHUNK,), jnp.int32),
        pltpu.VMEM((CHUNK,), jnp.int32),
        pltpu.VMEM((CHUNK,), jnp.float32),
        pltpu.VMEM((CHUNK,), jnp.float32),
        pltpu.VMEM((CHUNK,), jnp.float32),
        pltpu.VMEM((CHUNK,), jnp.float32),
        pltpu.SemaphoreType.DMA,
        pltpu.SemaphoreType.DMA,
        pltpu.SemaphoreType.DMA,
        pltpu.SemaphoreType.DMA,
        pltpu.SemaphoreType.DMA,
        pltpu.SemaphoreType.DMA,
    ],
)(_sc_scatter)


ROWS = 1_000
COLS = 1_600
BLK = 200


def _combine_body(ec_ref, cnt_ref, s_ref, r_ref, out_ref):
    ec = ec_ref[...]
    cnt_i = cnt_ref[...]
    cnt = jnp.maximum(cnt_i.astype(jnp.float32), 1.0)
    inv = 1.0 / cnt
    base = jnp.where(cnt_i > 0, 0.0, ec)
    out_ref[...] = base + s_ref[...] * inv + r_ref[...] * ec * inv * inv


def _combine(ec, cnt, s, r):
    grid = ROWS // BLK
    out = pl.pallas_call(
        _combine_body,
        out_shape=jax.ShapeDtypeStruct((ROWS, COLS), jnp.float32),
        grid=(grid,),
        in_specs=[
            pl.BlockSpec((BLK, COLS), lambda i: (i, 0)),
            pl.BlockSpec((BLK, COLS), lambda i: (i, 0)),
            pl.BlockSpec((BLK, COLS), lambda i: (i, 0)),
            pl.BlockSpec((BLK, COLS), lambda i: (i, 0)),
        ],
        out_specs=pl.BlockSpec((BLK, COLS), lambda i: (i, 0)),
    )(ec.reshape(ROWS, COLS), cnt.reshape(ROWS, COLS),
      s.reshape(ROWS, COLS), r.reshape(ROWS, COLS))
    return out.reshape(E)


def kernel(edge_costs, edge_counter, t12_costs, t13_costs, t23_costs,
           tri_corr_12, tri_corr_13, tri_corr_23, W1, b1, W2, b2):
    s, r = _sc_call(tri_corr_12, tri_corr_13, tri_corr_23,
                    t12_costs, t13_costs, t23_costs)
    ec = _combine(edge_costs, edge_counter, s, r)
    z = jnp.zeros_like(t12_costs)
    return ec, z, z, z
